# Initial kernel scaffold; baseline (speedup 1.0000x reference)
import jax
import jax.numpy as jnp
from jax.experimental import pallas as pl


def kernel(edge_index, edge_weight, user_emb_w, item_emb_w):
    raise NotImplementedError("write your pallas kernel here")


# speedup vs baseline: 7.3633x; 7.3633x over previous
import functools

import jax
import jax.numpy as jnp
from jax import lax
from jax.experimental import pallas as pl
from jax.experimental.pallas import tpu as pltpu
from jax.experimental.pallas import tpu_sc as plsc

_N_USERS = 25000
_N_ITEMS = 25000
_N = _N_USERS + _N_ITEMS
_EMB = 32
_E = 1600000

_NC = 2
_NS = 16
_NW = _NC * _NS

_CH = 128
_ROWS_PER_TILE = 400
_E_PAD = _NW * _ROWS_PER_TILE * _CH
_SUP = 16
_NSUP = _ROWS_PER_TILE // _SUP
_ZROWS = 80
_NZBLK = _N // _ZROWS
_ZBLK_PER_TILE = 40

_mesh = plsc.VectorSubcoreMesh(core_axis_name="c", subcore_axis_name="s")


@functools.partial(
    pl.kernel,
    out_type=[
        jax.ShapeDtypeStruct((_N, _EMB), jnp.float32),
        jax.ShapeDtypeStruct((_N, _EMB), jnp.float32),
    ],
    mesh=_mesh,
    compiler_params=pltpu.CompilerParams(use_tc_tiling_on_sc=False),
    scratch_types=[
        pltpu.VMEM_SHARED((_N, _EMB), jnp.float32),
        pltpu.VMEM((_SUP, _CH), jnp.int32),
        pltpu.VMEM((_SUP, _CH), jnp.int32),
        pltpu.VMEM((_SUP, _CH), jnp.float32),
        pltpu.VMEM((_ZROWS, _EMB), jnp.float32),
        pltpu.VMEM((_CH, _EMB), jnp.float32),
        pltpu.VMEM((_CH, _EMB), jnp.float32),
        pltpu.SemaphoreType.DMA,
    ],
)
def _layer(src_hbm, dst_hbm, w_hbm, table_hbm, out0, out1,
           acc, src_sc, dst_sc, w_sc, zbuf, gbuf, sbuf, gsem):
    cid = lax.axis_index("c")
    sid = lax.axis_index("s")

    zero = jnp.zeros((16,), jnp.float32)

    def _zrow(r, carry):
        zbuf[r, pl.ds(0, 16)] = zero
        zbuf[r, pl.ds(16, 16)] = zero
        return carry

    lax.fori_loop(0, _ZROWS, _zrow, 0)
    for k in range(_ZBLK_PER_TILE):
        b = sid + _NS * k

        @pl.when(b < _NZBLK)
        def _():
            pltpu.sync_copy(zbuf, acc.at[pl.ds(b * _ZROWS, _ZROWS)])
    plsc.subcore_barrier()

    wid = sid * _NC + cid
    row0 = wid * _ROWS_PER_TILE
    for sup in range(_NSUP):
        srow = row0 + sup * _SUP
        pltpu.sync_copy(src_hbm.at[pl.ds(srow, _SUP)], src_sc)
        pltpu.sync_copy(dst_hbm.at[pl.ds(srow, _SUP)], dst_sc)
        pltpu.sync_copy(w_hbm.at[pl.ds(srow, _SUP)], w_sc)

        def _chunk(j, carry):
            pltpu.async_copy(table_hbm.at[src_sc.at[j]], gbuf, gsem).wait()

            def _blk(b, c2):
                wv = w_sc[j, pl.ds(b * 16, 16)]
                r0 = b * 16
                for e in range(16):
                    w = wv[e]
                    r = r0 + e
                    sbuf[r, pl.ds(0, 16)] = gbuf[r, pl.ds(0, 16)] * w
                    sbuf[r, pl.ds(16, 16)] = gbuf[r, pl.ds(16, 16)] * w
                return c2

            lax.fori_loop(0, _CH // 16, _blk, 0)
            pltpu.sync_copy(sbuf, acc.at[dst_sc.at[j]], add=True)
            return carry

        lax.fori_loop(0, _SUP, _chunk, 0)

    plsc.subcore_barrier()

    for k in range(_ZBLK_PER_TILE):
        b = sid + _NS * k

        @pl.when((b < _NZBLK) & (cid == 0))
        def _():
            pltpu.sync_copy(acc.at[pl.ds(b * _ZROWS, _ZROWS)],
                            out0.at[pl.ds(b * _ZROWS, _ZROWS)])

        @pl.when((b < _NZBLK) & (cid == 1))
        def _():
            pltpu.sync_copy(acc.at[pl.ds(b * _ZROWS, _ZROWS)],
                            out1.at[pl.ds(b * _ZROWS, _ZROWS)])


_BLK = 400
_NBLK = _N // _BLK
_BLK_PER_W = 4


@functools.partial(
    pl.kernel,
    out_type=jax.ShapeDtypeStruct((_N, _EMB), jnp.float32),
    mesh=_mesh,
    compiler_params=pltpu.CompilerParams(use_tc_tiling_on_sc=False),
    scratch_types=[
        pltpu.VMEM((_BLK, _EMB), jnp.float32),
        pltpu.VMEM((_BLK, _EMB), jnp.float32),
    ],
)
def _add2(a_hbm, b_hbm, out, abuf, bbuf):
    cid = lax.axis_index("c")
    sid = lax.axis_index("s")
    wid = sid * _NC + cid

    def _accum(r, c2):
        abuf[r, pl.ds(0, 16)] = abuf[r, pl.ds(0, 16)] + bbuf[r, pl.ds(0, 16)]
        abuf[r, pl.ds(16, 16)] = abuf[r, pl.ds(16, 16)] + bbuf[r, pl.ds(16, 16)]
        return c2

    for k in range(_BLK_PER_W):
        b = wid + _NW * k

        @pl.when(b < _NBLK)
        def _():
            off = b * _BLK
            pltpu.sync_copy(a_hbm.at[pl.ds(off, _BLK)], abuf)
            pltpu.sync_copy(b_hbm.at[pl.ds(off, _BLK)], bbuf)
            lax.fori_loop(0, _BLK, _accum, 0, unroll=4)
            pltpu.sync_copy(abuf, out.at[pl.ds(off, _BLK)])


@functools.partial(
    pl.kernel,
    out_type=jax.ShapeDtypeStruct((_N, _EMB), jnp.float32),
    mesh=_mesh,
    compiler_params=pltpu.CompilerParams(use_tc_tiling_on_sc=False),
    scratch_types=[
        pltpu.VMEM((_BLK, _EMB), jnp.float32),
        pltpu.VMEM((_BLK, _EMB), jnp.float32),
    ],
)
def _add4_mean(a_hbm, b_hbm, c_hbm, d_hbm, out, abuf, bbuf):
    cid = lax.axis_index("c")
    sid = lax.axis_index("s")
    wid = sid * _NC + cid
    third = jnp.float32(1.0 / 3.0)

    def _accum(r, c2):
        abuf[r, pl.ds(0, 16)] = abuf[r, pl.ds(0, 16)] + bbuf[r, pl.ds(0, 16)]
        abuf[r, pl.ds(16, 16)] = abuf[r, pl.ds(16, 16)] + bbuf[r, pl.ds(16, 16)]
        return c2

    def _scale(r, c2):
        abuf[r, pl.ds(0, 16)] = abuf[r, pl.ds(0, 16)] * third
        abuf[r, pl.ds(16, 16)] = abuf[r, pl.ds(16, 16)] * third
        return c2

    for k in range(_BLK_PER_W):
        b = wid + _NW * k

        @pl.when(b < _NBLK)
        def _():
            off = b * _BLK
            pltpu.sync_copy(a_hbm.at[pl.ds(off, _BLK)], abuf)
            pltpu.sync_copy(b_hbm.at[pl.ds(off, _BLK)], bbuf)
            lax.fori_loop(0, _BLK, _accum, 0, unroll=4)
            pltpu.sync_copy(c_hbm.at[pl.ds(off, _BLK)], bbuf)
            lax.fori_loop(0, _BLK, _accum, 0, unroll=4)
            pltpu.sync_copy(d_hbm.at[pl.ds(off, _BLK)], bbuf)
            lax.fori_loop(0, _BLK, _accum, 0, unroll=4)
            lax.fori_loop(0, _BLK, _scale, 0, unroll=4)
            pltpu.sync_copy(abuf, out.at[pl.ds(off, _BLK)])


def kernel(edge_index, edge_weight, user_emb_w, item_emb_w):
    all0 = jnp.concatenate([user_emb_w, item_emb_w], axis=0)
    pad = _E_PAD - _E
    src = jnp.concatenate([edge_index[0], jnp.zeros((pad,), jnp.int32)])
    dst = jnp.concatenate([edge_index[1], jnp.zeros((pad,), jnp.int32)])
    w = jnp.concatenate([edge_weight, jnp.zeros((pad,), jnp.float32)])
    src2d = src.reshape(_E_PAD // _CH, _CH)
    dst2d = dst.reshape(_E_PAD // _CH, _CH)
    w2d = w.reshape(_E_PAD // _CH, _CH)

    p0, p1 = _layer(src2d, dst2d, w2d, all0)
    emb1 = _add2(p0, p1)
    q0, q1 = _layer(src2d, dst2d, w2d, emb1)
    final = _add4_mean(all0, emb1, q0, q1)
    return final[:_N_USERS], final[_N_USERS:]


# speedup vs baseline: 10.2659x; 1.3942x over previous
import functools

import jax
import jax.numpy as jnp
from jax import lax
from jax.experimental import pallas as pl
from jax.experimental.pallas import tpu as pltpu
from jax.experimental.pallas import tpu_sc as plsc

_N_USERS = 25000
_N_ITEMS = 25000
_N = _N_USERS + _N_ITEMS
_EMB = 32
_E = 1600000

_NC = 2
_NS = 16
_NW = _NC * _NS

_CH = 128
_ROWS_PER_TILE = 400
_E_PAD = _NW * _ROWS_PER_TILE * _CH
_SUP = 16
_NSUP = _ROWS_PER_TILE // _SUP
_NPAIR = _SUP // 2
_ZROWS = 80
_NZBLK = _N // _ZROWS
_ZBLK_PER_TILE = 40

_mesh = plsc.VectorSubcoreMesh(core_axis_name="c", subcore_axis_name="s")


@functools.partial(
    pl.kernel,
    out_type=[
        jax.ShapeDtypeStruct((_N, _EMB), jnp.float32),
        jax.ShapeDtypeStruct((_N, _EMB), jnp.float32),
    ],
    mesh=_mesh,
    compiler_params=pltpu.CompilerParams(use_tc_tiling_on_sc=False),
    scratch_types=[
        pltpu.VMEM_SHARED((_N, _EMB), jnp.float32),
        pltpu.VMEM((_SUP, _CH), jnp.int32),
        pltpu.VMEM((_SUP, _CH), jnp.int32),
        pltpu.VMEM((_SUP, _CH), jnp.float32),
        pltpu.VMEM((_ZROWS, _EMB), jnp.float32),
        pltpu.VMEM((_CH, _EMB), jnp.float32),
        pltpu.VMEM((_CH, _EMB), jnp.float32),
        pltpu.VMEM((_CH, _EMB), jnp.float32),
        pltpu.VMEM((_CH, _EMB), jnp.float32),
        pltpu.SemaphoreType.DMA,
        pltpu.SemaphoreType.DMA,
        pltpu.SemaphoreType.DMA,
        pltpu.SemaphoreType.DMA,
        pltpu.SemaphoreType.DMA,
    ],
)
def _layer(src_hbm, dst_hbm, w_hbm, table_hbm, out0, out1,
           acc, src_sc, dst_sc, w_sc, zbuf, g0, g1, s0, s1,
           sg0, sg1, ss0, ss1, sm):
    cid = lax.axis_index("c")
    sid = lax.axis_index("s")

    zero = jnp.zeros((16,), jnp.float32)

    def _zrow(r, carry):
        zbuf[r, pl.ds(0, 16)] = zero
        zbuf[r, pl.ds(16, 16)] = zero
        return carry

    lax.fori_loop(0, _ZROWS, _zrow, 0)
    for k in range(_ZBLK_PER_TILE):
        b = sid + _NS * k

        @pl.when(b < _NZBLK)
        def _():
            pltpu.sync_copy(zbuf, acc.at[pl.ds(b * _ZROWS, _ZROWS)])
    plsc.subcore_barrier()

    wid = sid * _NC + cid
    row0 = wid * _ROWS_PER_TILE

    def _scale(j, gbuf, sbuf):
        def _blk(b, c2):
            wv = w_sc[j, pl.ds(b * 16, 16)]
            r0 = b * 16
            for e in range(16):
                w = wv[e]
                r = r0 + e
                sbuf[r, pl.ds(0, 16)] = gbuf[r, pl.ds(0, 16)] * w
                sbuf[r, pl.ds(16, 16)] = gbuf[r, pl.ds(16, 16)] * w
            return c2

        lax.fori_loop(0, _CH // 16, _blk, 0)

    def _g_start(j, gbuf, sem):
        pltpu.async_copy(table_hbm.at[src_sc.at[j]], gbuf, sem)

    def _g_wait(j, gbuf, sem):
        pltpu.make_async_copy(table_hbm.at[src_sc.at[j]], gbuf, sem).wait()

    def _s_start(j, sbuf, sem):
        pltpu.async_copy(sbuf, acc.at[dst_sc.at[j]], sem, add=True)

    def _s_wait(j, sbuf, sem):
        pltpu.make_async_copy(sbuf, acc.at[dst_sc.at[j]], sem).wait()

    def _sup_body(sup, carry):
        srow = row0 + sup * _SUP
        pltpu.async_copy(src_hbm.at[pl.ds(srow, _SUP)], src_sc, sm)
        pltpu.async_copy(dst_hbm.at[pl.ds(srow, _SUP)], dst_sc, sm)
        pltpu.async_copy(w_hbm.at[pl.ds(srow, _SUP)], w_sc, sm)
        pltpu.make_async_copy(src_hbm.at[pl.ds(srow, _SUP)], src_sc, sm).wait()
        pltpu.make_async_copy(dst_hbm.at[pl.ds(srow, _SUP)], dst_sc, sm).wait()
        pltpu.make_async_copy(w_hbm.at[pl.ds(srow, _SUP)], w_sc, sm).wait()

        _g_start(0, g0, sg0)
        _g_start(1, g1, sg1)
        _g_wait(0, g0, sg0)
        _scale(0, g0, s0)
        _g_start(2, g0, sg0)
        _s_start(0, s0, ss0)
        _g_wait(1, g1, sg1)
        _scale(1, g1, s1)
        _g_start(3, g1, sg1)
        _s_start(1, s1, ss1)

        def _pair(p, c2):
            j0 = 2 * p
            _g_wait(j0, g0, sg0)
            _s_wait(j0 - 2, s0, ss0)
            _scale(j0, g0, s0)
            _g_start(j0 + 2, g0, sg0)
            _s_start(j0, s0, ss0)
            j1 = j0 + 1
            _g_wait(j1, g1, sg1)
            _s_wait(j1 - 2, s1, ss1)
            _scale(j1, g1, s1)
            _g_start(j1 + 2, g1, sg1)
            _s_start(j1, s1, ss1)
            return c2

        lax.fori_loop(1, _NPAIR - 1, _pair, 0)

        jl = _SUP - 2
        _g_wait(jl, g0, sg0)
        _s_wait(jl - 2, s0, ss0)
        _scale(jl, g0, s0)
        _s_start(jl, s0, ss0)
        _g_wait(jl + 1, g1, sg1)
        _s_wait(jl - 1, s1, ss1)
        _scale(jl + 1, g1, s1)
        _s_start(jl + 1, s1, ss1)
        _s_wait(jl, s0, ss0)
        _s_wait(jl + 1, s1, ss1)
        return carry

    lax.fori_loop(0, _NSUP, _sup_body, 0)

    plsc.subcore_barrier()

    for k in range(_ZBLK_PER_TILE):
        b = sid + _NS * k

        @pl.when((b < _NZBLK) & (cid == 0))
        def _():
            pltpu.sync_copy(acc.at[pl.ds(b * _ZROWS, _ZROWS)],
                            out0.at[pl.ds(b * _ZROWS, _ZROWS)])

        @pl.when((b < _NZBLK) & (cid == 1))
        def _():
            pltpu.sync_copy(acc.at[pl.ds(b * _ZROWS, _ZROWS)],
                            out1.at[pl.ds(b * _ZROWS, _ZROWS)])


_BLK = 400
_NBLK = _N // _BLK
_BLK_PER_W = 4


@functools.partial(
    pl.kernel,
    out_type=jax.ShapeDtypeStruct((_N, _EMB), jnp.float32),
    mesh=_mesh,
    compiler_params=pltpu.CompilerParams(use_tc_tiling_on_sc=False),
    scratch_types=[
        pltpu.VMEM((_BLK, _EMB), jnp.float32),
        pltpu.VMEM((_BLK, _EMB), jnp.float32),
    ],
)
def _add2(a_hbm, b_hbm, out, abuf, bbuf):
    cid = lax.axis_index("c")
    sid = lax.axis_index("s")
    wid = sid * _NC + cid

    def _accum(r, c2):
        abuf[r, pl.ds(0, 16)] = abuf[r, pl.ds(0, 16)] + bbuf[r, pl.ds(0, 16)]
        abuf[r, pl.ds(16, 16)] = abuf[r, pl.ds(16, 16)] + bbuf[r, pl.ds(16, 16)]
        return c2

    for k in range(_BLK_PER_W):
        b = wid + _NW * k

        @pl.when(b < _NBLK)
        def _():
            off = b * _BLK
            pltpu.sync_copy(a_hbm.at[pl.ds(off, _BLK)], abuf)
            pltpu.sync_copy(b_hbm.at[pl.ds(off, _BLK)], bbuf)
            lax.fori_loop(0, _BLK, _accum, 0, unroll=4)
            pltpu.sync_copy(abuf, out.at[pl.ds(off, _BLK)])


@functools.partial(
    pl.kernel,
    out_type=jax.ShapeDtypeStruct((_N, _EMB), jnp.float32),
    mesh=_mesh,
    compiler_params=pltpu.CompilerParams(use_tc_tiling_on_sc=False),
    scratch_types=[
        pltpu.VMEM((_BLK, _EMB), jnp.float32),
        pltpu.VMEM((_BLK, _EMB), jnp.float32),
    ],
)
def _add4_mean(a_hbm, b_hbm, c_hbm, d_hbm, out, abuf, bbuf):
    cid = lax.axis_index("c")
    sid = lax.axis_index("s")
    wid = sid * _NC + cid
    third = jnp.float32(1.0 / 3.0)

    def _accum(r, c2):
        abuf[r, pl.ds(0, 16)] = abuf[r, pl.ds(0, 16)] + bbuf[r, pl.ds(0, 16)]
        abuf[r, pl.ds(16, 16)] = abuf[r, pl.ds(16, 16)] + bbuf[r, pl.ds(16, 16)]
        return c2

    def _scale(r, c2):
        abuf[r, pl.ds(0, 16)] = abuf[r, pl.ds(0, 16)] * third
        abuf[r, pl.ds(16, 16)] = abuf[r, pl.ds(16, 16)] * third
        return c2

    for k in range(_BLK_PER_W):
        b = wid + _NW * k

        @pl.when(b < _NBLK)
        def _():
            off = b * _BLK
            pltpu.sync_copy(a_hbm.at[pl.ds(off, _BLK)], abuf)
            pltpu.sync_copy(b_hbm.at[pl.ds(off, _BLK)], bbuf)
            lax.fori_loop(0, _BLK, _accum, 0, unroll=4)
            pltpu.sync_copy(c_hbm.at[pl.ds(off, _BLK)], bbuf)
            lax.fori_loop(0, _BLK, _accum, 0, unroll=4)
            pltpu.sync_copy(d_hbm.at[pl.ds(off, _BLK)], bbuf)
            lax.fori_loop(0, _BLK, _accum, 0, unroll=4)
            lax.fori_loop(0, _BLK, _scale, 0, unroll=4)
            pltpu.sync_copy(abuf, out.at[pl.ds(off, _BLK)])


def kernel(edge_index, edge_weight, user_emb_w, item_emb_w):
    all0 = jnp.concatenate([user_emb_w, item_emb_w], axis=0)
    pad = _E_PAD - _E
    src = jnp.concatenate([edge_index[0], jnp.zeros((pad,), jnp.int32)])
    dst = jnp.concatenate([edge_index[1], jnp.zeros((pad,), jnp.int32)])
    w = jnp.concatenate([edge_weight, jnp.zeros((pad,), jnp.float32)])
    src2d = src.reshape(_E_PAD // _CH, _CH)
    dst2d = dst.reshape(_E_PAD // _CH, _CH)
    w2d = w.reshape(_E_PAD // _CH, _CH)

    p0, p1 = _layer(src2d, dst2d, w2d, all0)
    emb1 = _add2(p0, p1)
    q0, q1 = _layer(src2d, dst2d, w2d, emb1)
    final = _add4_mean(all0, emb1, q0, q1)
    return final[:_N_USERS], final[_N_USERS:]


# speedup vs baseline: 12.5917x; 1.2266x over previous
import functools

import jax
import jax.numpy as jnp
from jax import lax
from jax.experimental import pallas as pl
from jax.experimental.pallas import tpu as pltpu
from jax.experimental.pallas import tpu_sc as plsc

_N_USERS = 25000
_N_ITEMS = 25000
_N = _N_USERS + _N_ITEMS
_EMB = 32
_E = 1600000

_NC = 2
_NS = 16
_NW = _NC * _NS

_CH = 128
_ROWS_PER_TILE = 400
_E_PAD = _NW * _ROWS_PER_TILE * _CH
_SUP = 16
_NSUP = _ROWS_PER_TILE // _SUP
_NPAIR = _SUP // 2
_ZROWS = 80
_NZBLK = _N // _ZROWS
_ZBLK_PER_TILE = 40

_mesh = plsc.VectorSubcoreMesh(core_axis_name="c", subcore_axis_name="s")


@functools.partial(
    pl.kernel,
    out_type=[
        jax.ShapeDtypeStruct((_N, _EMB), jnp.float32),
        jax.ShapeDtypeStruct((_N, _EMB), jnp.float32),
    ],
    mesh=_mesh,
    compiler_params=pltpu.CompilerParams(use_tc_tiling_on_sc=False, needs_layout_passes=False),
    scratch_types=[
        pltpu.VMEM_SHARED((_N, _EMB), jnp.float32),
        pltpu.VMEM((_SUP, _CH), jnp.int32),
        pltpu.VMEM((_SUP, _CH), jnp.int32),
        pltpu.VMEM((_SUP, _CH), jnp.float32),
        pltpu.VMEM((_ZROWS, _EMB), jnp.float32),
        pltpu.VMEM((_CH, _EMB), jnp.bfloat16),
        pltpu.VMEM((_CH, _EMB), jnp.bfloat16),
        pltpu.VMEM((_CH, _EMB), jnp.float32),
        pltpu.VMEM((_CH, _EMB), jnp.float32),
        pltpu.SemaphoreType.DMA,
        pltpu.SemaphoreType.DMA,
        pltpu.SemaphoreType.DMA,
        pltpu.SemaphoreType.DMA,
        pltpu.SemaphoreType.DMA,
    ],
)
def _layer(src_hbm, dst_hbm, w_hbm, table_hbm, out0, out1,
           acc, src_sc, dst_sc, w_sc, zbuf, g0, g1, s0, s1,
           sg0, sg1, ss0, ss1, sm):
    cid = lax.axis_index("c")
    sid = lax.axis_index("s")

    zero = jnp.zeros((16,), jnp.float32)

    def _zrow(r, carry):
        zbuf[r, pl.ds(0, 16)] = zero
        zbuf[r, pl.ds(16, 16)] = zero
        return carry

    lax.fori_loop(0, _ZROWS, _zrow, 0)
    for k in range(_ZBLK_PER_TILE):
        b = sid + _NS * k

        @pl.when(b < _NZBLK)
        def _():
            pltpu.sync_copy(zbuf, acc.at[pl.ds(b * _ZROWS, _ZROWS)])
    plsc.subcore_barrier()

    wid = sid * _NC + cid
    row0 = wid * _ROWS_PER_TILE

    def _scale(j, gbuf, sbuf):
        def _blk(b, c2):
            wv = w_sc[j, pl.ds(b * 16, 16)]
            r0 = b * 16
            for e in range(16):
                w = wv[e]
                r = r0 + e
                lo, hi = plsc.unpack(gbuf[r, pl.ds(0, _EMB)],
                                     format=plsc.PackFormat.INTERLEAVED)
                sbuf[r, pl.ds(0, 16)] = lo * w
                sbuf[r, pl.ds(16, 16)] = hi * w
            return c2

        lax.fori_loop(0, _CH // 16, _blk, 0)

    def _g_start(j, gbuf, sem):
        pltpu.async_copy(table_hbm.at[src_sc.at[j]], gbuf, sem)

    def _g_wait(j, gbuf, sem):
        pltpu.make_async_copy(table_hbm.at[src_sc.at[j]], gbuf, sem).wait()

    def _s_start(j, sbuf, sem):
        pltpu.async_copy(sbuf, acc.at[dst_sc.at[j]], sem, add=True)

    def _s_wait(j, sbuf, sem):
        pltpu.make_async_copy(sbuf, acc.at[dst_sc.at[j]], sem).wait()

    def _sup_body(sup, carry):
        srow = row0 + sup * _SUP
        pltpu.async_copy(src_hbm.at[pl.ds(srow, _SUP)], src_sc, sm)
        pltpu.async_copy(dst_hbm.at[pl.ds(srow, _SUP)], dst_sc, sm)
        pltpu.async_copy(w_hbm.at[pl.ds(srow, _SUP)], w_sc, sm)
        pltpu.make_async_copy(src_hbm.at[pl.ds(srow, _SUP)], src_sc, sm).wait()
        pltpu.make_async_copy(dst_hbm.at[pl.ds(srow, _SUP)], dst_sc, sm).wait()
        pltpu.make_async_copy(w_hbm.at[pl.ds(srow, _SUP)], w_sc, sm).wait()

        _g_start(0, g0, sg0)
        _g_start(1, g1, sg1)
        _g_wait(0, g0, sg0)
        _scale(0, g0, s0)
        _g_start(2, g0, sg0)
        _s_start(0, s0, ss0)
        _g_wait(1, g1, sg1)
        _scale(1, g1, s1)
        _g_start(3, g1, sg1)
        _s_start(1, s1, ss1)

        def _pair(p, c2):
            j0 = 2 * p
            _g_wait(j0, g0, sg0)
            _s_wait(j0 - 2, s0, ss0)
            _scale(j0, g0, s0)
            _g_start(j0 + 2, g0, sg0)
            _s_start(j0, s0, ss0)
            j1 = j0 + 1
            _g_wait(j1, g1, sg1)
            _s_wait(j1 - 2, s1, ss1)
            _scale(j1, g1, s1)
            _g_start(j1 + 2, g1, sg1)
            _s_start(j1, s1, ss1)
            return c2

        lax.fori_loop(1, _NPAIR - 1, _pair, 0)

        jl = _SUP - 2
        _g_wait(jl, g0, sg0)
        _s_wait(jl - 2, s0, ss0)
        _scale(jl, g0, s0)
        _s_start(jl, s0, ss0)
        _g_wait(jl + 1, g1, sg1)
        _s_wait(jl - 1, s1, ss1)
        _scale(jl + 1, g1, s1)
        _s_start(jl + 1, s1, ss1)
        _s_wait(jl, s0, ss0)
        _s_wait(jl + 1, s1, ss1)
        return carry

    lax.fori_loop(0, _NSUP, _sup_body, 0)

    plsc.subcore_barrier()

    for k in range(_ZBLK_PER_TILE):
        b = sid + _NS * k

        @pl.when((b < _NZBLK) & (cid == 0))
        def _():
            pltpu.sync_copy(acc.at[pl.ds(b * _ZROWS, _ZROWS)],
                            out0.at[pl.ds(b * _ZROWS, _ZROWS)])

        @pl.when((b < _NZBLK) & (cid == 1))
        def _():
            pltpu.sync_copy(acc.at[pl.ds(b * _ZROWS, _ZROWS)],
                            out1.at[pl.ds(b * _ZROWS, _ZROWS)])


_BLK = 400
_NBLK = _N // _BLK
_BLK_PER_W = 4


@functools.partial(
    pl.kernel,
    out_type=[
        jax.ShapeDtypeStruct((_N, _EMB), jnp.float32),
        jax.ShapeDtypeStruct((_N, _EMB), jnp.bfloat16),
    ],
    mesh=_mesh,
    compiler_params=pltpu.CompilerParams(use_tc_tiling_on_sc=False, needs_layout_passes=False),
    scratch_types=[
        pltpu.VMEM((_BLK, _EMB), jnp.float32),
        pltpu.VMEM((_BLK, _EMB), jnp.float32),
        pltpu.VMEM((_BLK, _EMB), jnp.bfloat16),
    ],
)
def _add2(a_hbm, b_hbm, out, out_bf, abuf, bbuf, pbuf):
    cid = lax.axis_index("c")
    sid = lax.axis_index("s")
    wid = sid * _NC + cid

    def _accum(r, c2):
        lo = abuf[r, pl.ds(0, 16)] + bbuf[r, pl.ds(0, 16)]
        hi = abuf[r, pl.ds(16, 16)] + bbuf[r, pl.ds(16, 16)]
        abuf[r, pl.ds(0, 16)] = lo
        abuf[r, pl.ds(16, 16)] = hi
        pbuf[r, pl.ds(0, _EMB)] = plsc.pack(
            lo, hi, format=plsc.PackFormat.INTERLEAVED)
        return c2

    for k in range(_BLK_PER_W):
        b = wid + _NW * k

        @pl.when(b < _NBLK)
        def _():
            off = b * _BLK
            pltpu.sync_copy(a_hbm.at[pl.ds(off, _BLK)], abuf)
            pltpu.sync_copy(b_hbm.at[pl.ds(off, _BLK)], bbuf)
            lax.fori_loop(0, _BLK, _accum, 0, unroll=4)
            pltpu.sync_copy(abuf, out.at[pl.ds(off, _BLK)])
            pltpu.sync_copy(pbuf, out_bf.at[pl.ds(off, _BLK)])


@functools.partial(
    pl.kernel,
    out_type=jax.ShapeDtypeStruct((_N, _EMB), jnp.float32),
    mesh=_mesh,
    compiler_params=pltpu.CompilerParams(use_tc_tiling_on_sc=False, needs_layout_passes=False),
    scratch_types=[
        pltpu.VMEM((_BLK, _EMB), jnp.float32),
        pltpu.VMEM((_BLK, _EMB), jnp.float32),
    ],
)
def _add4_mean(a_hbm, b_hbm, c_hbm, d_hbm, out, abuf, bbuf):
    cid = lax.axis_index("c")
    sid = lax.axis_index("s")
    wid = sid * _NC + cid
    third = jnp.float32(1.0 / 3.0)

    def _accum(r, c2):
        abuf[r, pl.ds(0, 16)] = abuf[r, pl.ds(0, 16)] + bbuf[r, pl.ds(0, 16)]
        abuf[r, pl.ds(16, 16)] = abuf[r, pl.ds(16, 16)] + bbuf[r, pl.ds(16, 16)]
        return c2

    def _scale(r, c2):
        abuf[r, pl.ds(0, 16)] = abuf[r, pl.ds(0, 16)] * third
        abuf[r, pl.ds(16, 16)] = abuf[r, pl.ds(16, 16)] * third
        return c2

    for k in range(_BLK_PER_W):
        b = wid + _NW * k

        @pl.when(b < _NBLK)
        def _():
            off = b * _BLK
            pltpu.sync_copy(a_hbm.at[pl.ds(off, _BLK)], abuf)
            pltpu.sync_copy(b_hbm.at[pl.ds(off, _BLK)], bbuf)
            lax.fori_loop(0, _BLK, _accum, 0, unroll=4)
            pltpu.sync_copy(c_hbm.at[pl.ds(off, _BLK)], bbuf)
            lax.fori_loop(0, _BLK, _accum, 0, unroll=4)
            pltpu.sync_copy(d_hbm.at[pl.ds(off, _BLK)], bbuf)
            lax.fori_loop(0, _BLK, _accum, 0, unroll=4)
            lax.fori_loop(0, _BLK, _scale, 0, unroll=4)
            pltpu.sync_copy(abuf, out.at[pl.ds(off, _BLK)])


def _to_packed_bf16(x):
    lo = x[:, : _EMB // 2]
    hi = x[:, _EMB // 2:]
    return jnp.stack([lo, hi], axis=-1).reshape(_N, _EMB).astype(jnp.bfloat16)


def kernel(edge_index, edge_weight, user_emb_w, item_emb_w):
    all0 = jnp.concatenate([user_emb_w, item_emb_w], axis=0)
    pad = _E_PAD - _E
    src = jnp.concatenate([edge_index[0], jnp.zeros((pad,), jnp.int32)])
    dst = jnp.concatenate([edge_index[1], jnp.zeros((pad,), jnp.int32)])
    w = jnp.concatenate([edge_weight, jnp.zeros((pad,), jnp.float32)])
    src2d = src.reshape(_E_PAD // _CH, _CH)
    dst2d = dst.reshape(_E_PAD // _CH, _CH)
    w2d = w.reshape(_E_PAD // _CH, _CH)

    p0, p1 = _layer(src2d, dst2d, w2d, _to_packed_bf16(all0))
    emb1, emb1_bf = _add2(p0, p1)
    q0, q1 = _layer(src2d, dst2d, w2d, emb1_bf)
    final = _add4_mean(all0, emb1, q0, q1)
    return final[:_N_USERS], final[_N_USERS:]


# speedup vs baseline: 14.0172x; 1.1132x over previous
import functools

import jax
import jax.numpy as jnp
from jax import lax
from jax.experimental import pallas as pl
from jax.experimental.pallas import tpu as pltpu
from jax.experimental.pallas import tpu_sc as plsc

_N_USERS = 25000
_N_ITEMS = 25000
_N = _N_USERS + _N_ITEMS
_EMB = 32
_E = 1600000

_NC = 2
_NS = 16
_NW = _NC * _NS

_CH = 128
_ROWS_PER_TILE = 396
_E_PAD = _NW * _ROWS_PER_TILE * _CH
_SUP = 18
_NSUP = _ROWS_PER_TILE // _SUP
_NTRI = _SUP // 3
_ZROWS = 80
_NZBLK = _N // _ZROWS
_ZBLK_PER_TILE = 40

_mesh = plsc.VectorSubcoreMesh(core_axis_name="c", subcore_axis_name="s")


@functools.partial(
    pl.kernel,
    out_type=[
        jax.ShapeDtypeStruct((_N, _EMB), jnp.float32),
        jax.ShapeDtypeStruct((_N, _EMB), jnp.float32),
    ],
    mesh=_mesh,
    compiler_params=pltpu.CompilerParams(use_tc_tiling_on_sc=False, needs_layout_passes=False),
    scratch_types=[
        pltpu.VMEM_SHARED((_N, _EMB), jnp.float32),
        pltpu.VMEM((_SUP, _CH), jnp.int32),
        pltpu.VMEM((_SUP, _CH), jnp.int32),
        pltpu.VMEM((_SUP, _CH), jnp.float32),
        pltpu.VMEM((_ZROWS, _EMB), jnp.float32),
        pltpu.VMEM((_CH, _EMB), jnp.bfloat16),
        pltpu.VMEM((_CH, _EMB), jnp.bfloat16),
        pltpu.VMEM((_CH, _EMB), jnp.bfloat16),
        pltpu.VMEM((_CH, _EMB), jnp.float32),
        pltpu.VMEM((_CH, _EMB), jnp.float32),
        pltpu.VMEM((_CH, _EMB), jnp.float32),
        pltpu.SemaphoreType.DMA,
        pltpu.SemaphoreType.DMA,
        pltpu.SemaphoreType.DMA,
        pltpu.SemaphoreType.DMA,
        pltpu.SemaphoreType.DMA,
        pltpu.SemaphoreType.DMA,
        pltpu.SemaphoreType.DMA,
    ],
)
def _layer(src_hbm, dst_hbm, w_hbm, table_hbm, out0, out1,
           acc, src_sc, dst_sc, w_sc, zbuf, g0, g1, g2, s0, s1, s2,
           sg0, sg1, sg2, ss0, ss1, ss2, sm):
    cid = lax.axis_index("c")
    sid = lax.axis_index("s")

    zero = jnp.zeros((16,), jnp.float32)

    def _zrow(r, carry):
        zbuf[r, pl.ds(0, 16)] = zero
        zbuf[r, pl.ds(16, 16)] = zero
        return carry

    lax.fori_loop(0, _ZROWS, _zrow, 0)
    for k in range(_ZBLK_PER_TILE):
        b = sid + _NS * k

        @pl.when(b < _NZBLK)
        def _():
            pltpu.sync_copy(zbuf, acc.at[pl.ds(b * _ZROWS, _ZROWS)])
    plsc.subcore_barrier()

    wid = sid * _NC + cid
    row0 = wid * _ROWS_PER_TILE

    def _scale(j, gbuf, sbuf):
        def _blk(b, c2):
            wv = w_sc[j, pl.ds(b * 16, 16)]
            r0 = b * 16
            for e in range(16):
                w = wv[e]
                r = r0 + e
                lo, hi = plsc.unpack(gbuf[r, pl.ds(0, _EMB)],
                                     format=plsc.PackFormat.INTERLEAVED)
                sbuf[r, pl.ds(0, 16)] = lo * w
                sbuf[r, pl.ds(16, 16)] = hi * w
            return c2

        lax.fori_loop(0, _CH // 16, _blk, 0)

    def _g_start(j, gbuf, sem):
        pltpu.async_copy(table_hbm.at[src_sc.at[j]], gbuf, sem)

    def _g_wait(j, gbuf, sem):
        pltpu.make_async_copy(table_hbm.at[src_sc.at[j]], gbuf, sem).wait()

    def _s_start(j, sbuf, sem):
        pltpu.async_copy(sbuf, acc.at[dst_sc.at[j]], sem, add=True)

    def _s_wait(j, sbuf, sem):
        pltpu.make_async_copy(sbuf, acc.at[dst_sc.at[j]], sem).wait()

    def _sup_body(sup, carry):
        srow = row0 + sup * _SUP
        pltpu.async_copy(src_hbm.at[pl.ds(srow, _SUP)], src_sc, sm)
        pltpu.async_copy(dst_hbm.at[pl.ds(srow, _SUP)], dst_sc, sm)
        pltpu.async_copy(w_hbm.at[pl.ds(srow, _SUP)], w_sc, sm)
        pltpu.make_async_copy(src_hbm.at[pl.ds(srow, _SUP)], src_sc, sm).wait()
        pltpu.make_async_copy(dst_hbm.at[pl.ds(srow, _SUP)], dst_sc, sm).wait()
        pltpu.make_async_copy(w_hbm.at[pl.ds(srow, _SUP)], w_sc, sm).wait()

        gbufs = ((g0, sg0), (g1, sg1), (g2, sg2))
        sbufs = ((s0, ss0), (s1, ss1), (s2, ss2))
        for u in range(3):
            _g_start(u, *gbufs[u])
        for u in range(3):
            gb, gs = gbufs[u]
            sb, ssem = sbufs[u]
            _g_wait(u, gb, gs)
            _scale(u, gb, sb)
            _g_start(u + 3, gb, gs)
            _s_start(u, sb, ssem)

        def _tri(t, c2):
            for u in range(3):
                j = 3 * t + u
                gb, gs = gbufs[u]
                sb, ssem = sbufs[u]
                _g_wait(j, gb, gs)
                _s_wait(j - 3, sb, ssem)
                _scale(j, gb, sb)
                _g_start(j + 3, gb, gs)
                _s_start(j, sb, ssem)
            return c2

        lax.fori_loop(1, _NTRI - 1, _tri, 0)

        jl = _SUP - 3
        for u in range(3):
            gb, gs = gbufs[u]
            sb, ssem = sbufs[u]
            _g_wait(jl + u, gb, gs)
            _s_wait(jl + u - 3, sb, ssem)
            _scale(jl + u, gb, sb)
            _s_start(jl + u, sb, ssem)
        for u in range(3):
            sb, ssem = sbufs[u]
            _s_wait(jl + u, sb, ssem)
        return carry

    lax.fori_loop(0, _NSUP, _sup_body, 0)

    plsc.subcore_barrier()

    for k in range(_ZBLK_PER_TILE):
        b = sid + _NS * k

        @pl.when((b < _NZBLK) & (cid == 0))
        def _():
            pltpu.sync_copy(acc.at[pl.ds(b * _ZROWS, _ZROWS)],
                            out0.at[pl.ds(b * _ZROWS, _ZROWS)])

        @pl.when((b < _NZBLK) & (cid == 1))
        def _():
            pltpu.sync_copy(acc.at[pl.ds(b * _ZROWS, _ZROWS)],
                            out1.at[pl.ds(b * _ZROWS, _ZROWS)])


_BLK = 400
_NBLK = _N // _BLK
_BLK_PER_W = 4


@functools.partial(
    pl.kernel,
    out_type=[
        jax.ShapeDtypeStruct((_N, _EMB), jnp.float32),
        jax.ShapeDtypeStruct((_N, _EMB), jnp.bfloat16),
    ],
    mesh=_mesh,
    compiler_params=pltpu.CompilerParams(use_tc_tiling_on_sc=False, needs_layout_passes=False),
    scratch_types=[
        pltpu.VMEM((_BLK, _EMB), jnp.float32),
        pltpu.VMEM((_BLK, _EMB), jnp.float32),
        pltpu.VMEM((_BLK, _EMB), jnp.bfloat16),
    ],
)
def _add2(a_hbm, b_hbm, out, out_bf, abuf, bbuf, pbuf):
    cid = lax.axis_index("c")
    sid = lax.axis_index("s")
    wid = sid * _NC + cid

    def _accum(r, c2):
        lo = abuf[r, pl.ds(0, 16)] + bbuf[r, pl.ds(0, 16)]
        hi = abuf[r, pl.ds(16, 16)] + bbuf[r, pl.ds(16, 16)]
        abuf[r, pl.ds(0, 16)] = lo
        abuf[r, pl.ds(16, 16)] = hi
        pbuf[r, pl.ds(0, _EMB)] = plsc.pack(
            lo, hi, format=plsc.PackFormat.INTERLEAVED)
        return c2

    for k in range(_BLK_PER_W):
        b = wid + _NW * k

        @pl.when(b < _NBLK)
        def _():
            off = b * _BLK
            pltpu.sync_copy(a_hbm.at[pl.ds(off, _BLK)], abuf)
            pltpu.sync_copy(b_hbm.at[pl.ds(off, _BLK)], bbuf)
            lax.fori_loop(0, _BLK, _accum, 0, unroll=4)
            pltpu.sync_copy(abuf, out.at[pl.ds(off, _BLK)])
            pltpu.sync_copy(pbuf, out_bf.at[pl.ds(off, _BLK)])


@functools.partial(
    pl.kernel,
    out_type=jax.ShapeDtypeStruct((_N, _EMB), jnp.float32),
    mesh=_mesh,
    compiler_params=pltpu.CompilerParams(use_tc_tiling_on_sc=False, needs_layout_passes=False),
    scratch_types=[
        pltpu.VMEM((_BLK, _EMB), jnp.float32),
        pltpu.VMEM((_BLK, _EMB), jnp.float32),
    ],
)
def _add4_mean(a_hbm, b_hbm, c_hbm, d_hbm, out, abuf, bbuf):
    cid = lax.axis_index("c")
    sid = lax.axis_index("s")
    wid = sid * _NC + cid
    third = jnp.float32(1.0 / 3.0)

    def _accum(r, c2):
        abuf[r, pl.ds(0, 16)] = abuf[r, pl.ds(0, 16)] + bbuf[r, pl.ds(0, 16)]
        abuf[r, pl.ds(16, 16)] = abuf[r, pl.ds(16, 16)] + bbuf[r, pl.ds(16, 16)]
        return c2

    def _scale(r, c2):
        abuf[r, pl.ds(0, 16)] = abuf[r, pl.ds(0, 16)] * third
        abuf[r, pl.ds(16, 16)] = abuf[r, pl.ds(16, 16)] * third
        return c2

    for k in range(_BLK_PER_W):
        b = wid + _NW * k

        @pl.when(b < _NBLK)
        def _():
            off = b * _BLK
            pltpu.sync_copy(a_hbm.at[pl.ds(off, _BLK)], abuf)
            pltpu.sync_copy(b_hbm.at[pl.ds(off, _BLK)], bbuf)
            lax.fori_loop(0, _BLK, _accum, 0, unroll=4)
            pltpu.sync_copy(c_hbm.at[pl.ds(off, _BLK)], bbuf)
            lax.fori_loop(0, _BLK, _accum, 0, unroll=4)
            pltpu.sync_copy(d_hbm.at[pl.ds(off, _BLK)], bbuf)
            lax.fori_loop(0, _BLK, _accum, 0, unroll=4)
            lax.fori_loop(0, _BLK, _scale, 0, unroll=4)
            pltpu.sync_copy(abuf, out.at[pl.ds(off, _BLK)])


def _to_packed_bf16(x):
    lo = x[:, : _EMB // 2]
    hi = x[:, _EMB // 2:]
    return jnp.stack([lo, hi], axis=-1).reshape(_N, _EMB).astype(jnp.bfloat16)


def kernel(edge_index, edge_weight, user_emb_w, item_emb_w):
    all0 = jnp.concatenate([user_emb_w, item_emb_w], axis=0)
    pad = _E_PAD - _E
    src = jnp.concatenate([edge_index[0], jnp.zeros((pad,), jnp.int32)])
    dst = jnp.concatenate([edge_index[1], jnp.zeros((pad,), jnp.int32)])
    w = jnp.concatenate([edge_weight, jnp.zeros((pad,), jnp.float32)])
    src2d = src.reshape(_E_PAD // _CH, _CH)
    dst2d = dst.reshape(_E_PAD // _CH, _CH)
    w2d = w.reshape(_E_PAD // _CH, _CH)

    p0, p1 = _layer(src2d, dst2d, w2d, _to_packed_bf16(all0))
    emb1, emb1_bf = _add2(p0, p1)
    q0, q1 = _layer(src2d, dst2d, w2d, emb1_bf)
    final = _add4_mean(all0, emb1, q0, q1)
    return final[:_N_USERS], final[_N_USERS:]


# speedup vs baseline: 15.1359x; 1.0798x over previous
import functools

import jax
import jax.numpy as jnp
from jax import lax
from jax.experimental import pallas as pl
from jax.experimental.pallas import tpu as pltpu
from jax.experimental.pallas import tpu_sc as plsc

_N_USERS = 25000
_N_ITEMS = 25000
_N = _N_USERS + _N_ITEMS
_EMB = 32
_E = 1600000

_NC = 2
_NS = 16
_NW = _NC * _NS

_CH = 128
_ROWS_PER_TILE = 396
_E_PAD = _NW * _ROWS_PER_TILE * _CH
_SUP = 18
_NSUP = _ROWS_PER_TILE // _SUP
_NTRI = _SUP // 3
_ZROWS = 80
_NZBLK = _N // _ZROWS
_ZBLK_PER_TILE = 40

_mesh = plsc.VectorSubcoreMesh(core_axis_name="c", subcore_axis_name="s")


@functools.partial(
    pl.kernel,
    out_type=[
        jax.ShapeDtypeStruct((_N, _EMB), jnp.float32),
        jax.ShapeDtypeStruct((_N, _EMB), jnp.float32),
    ],
    mesh=_mesh,
    compiler_params=pltpu.CompilerParams(use_tc_tiling_on_sc=False, needs_layout_passes=False),
    scratch_types=[
        pltpu.VMEM_SHARED((_N, _EMB), jnp.float32),
        pltpu.VMEM((_SUP, _CH), jnp.int32),
        pltpu.VMEM((_SUP, _CH), jnp.int32),
        pltpu.VMEM((_SUP, _CH), jnp.float32),
        pltpu.VMEM((_ZROWS, _EMB), jnp.float32),
        pltpu.VMEM((_CH, _EMB), jnp.bfloat16),
        pltpu.VMEM((_CH, _EMB), jnp.bfloat16),
        pltpu.VMEM((_CH, _EMB), jnp.bfloat16),
        pltpu.VMEM((_CH, _EMB), jnp.float32),
        pltpu.VMEM((_CH, _EMB), jnp.float32),
        pltpu.VMEM((_CH, _EMB), jnp.float32),
        pltpu.SemaphoreType.DMA,
        pltpu.SemaphoreType.DMA,
        pltpu.SemaphoreType.DMA,
        pltpu.SemaphoreType.DMA,
        pltpu.SemaphoreType.DMA,
        pltpu.SemaphoreType.DMA,
        pltpu.SemaphoreType.DMA,
    ],
)
def _layer(src_hbm, dst_hbm, w_hbm, table_hbm, out0, out1,
           acc, src_sc, dst_sc, w_sc, zbuf, g0, g1, g2, s0, s1, s2,
           sg0, sg1, sg2, ss0, ss1, ss2, sm):
    cid = lax.axis_index("c")
    sid = lax.axis_index("s")

    zero = jnp.zeros((16,), jnp.float32)

    def _zrow(r, carry):
        zbuf[r, pl.ds(0, 16)] = zero
        zbuf[r, pl.ds(16, 16)] = zero
        return carry

    lax.fori_loop(0, _ZROWS, _zrow, 0)
    for k in range(_ZBLK_PER_TILE):
        b = sid + _NS * k

        @pl.when(b < _NZBLK)
        def _():
            pltpu.async_copy(zbuf, acc.at[pl.ds(b * _ZROWS, _ZROWS)], sm)
    for k in range(_ZBLK_PER_TILE):
        b = sid + _NS * k

        @pl.when(b < _NZBLK)
        def _():
            pltpu.make_async_copy(zbuf, acc.at[pl.ds(b * _ZROWS, _ZROWS)],
                                  sm).wait()
    plsc.subcore_barrier()

    wid = sid * _NC + cid
    row0 = wid * _ROWS_PER_TILE

    def _scale(j, gbuf, sbuf):
        def _blk(b, c2):
            wv = w_sc[j, pl.ds(b * 16, 16)]
            r0 = b * 16
            for e in range(16):
                w = wv[e]
                r = r0 + e
                lo, hi = plsc.unpack(gbuf[r, pl.ds(0, _EMB)],
                                     format=plsc.PackFormat.INTERLEAVED)
                sbuf[r, pl.ds(0, 16)] = lo * w
                sbuf[r, pl.ds(16, 16)] = hi * w
            return c2

        lax.fori_loop(0, _CH // 16, _blk, 0)

    def _g_start(j, gbuf, sem):
        pltpu.async_copy(table_hbm.at[src_sc.at[j]], gbuf, sem)

    def _g_wait(j, gbuf, sem):
        pltpu.make_async_copy(table_hbm.at[src_sc.at[j]], gbuf, sem).wait()

    def _s_start(j, sbuf, sem):
        pltpu.async_copy(sbuf, acc.at[dst_sc.at[j]], sem, add=True)

    def _s_wait(j, sbuf, sem):
        pltpu.make_async_copy(sbuf, acc.at[dst_sc.at[j]], sem).wait()

    def _sup_body(sup, carry):
        srow = row0 + sup * _SUP
        pltpu.async_copy(src_hbm.at[pl.ds(srow, _SUP)], src_sc, sm)
        pltpu.async_copy(dst_hbm.at[pl.ds(srow, _SUP)], dst_sc, sm)
        pltpu.async_copy(w_hbm.at[pl.ds(srow, _SUP)], w_sc, sm)
        pltpu.make_async_copy(src_hbm.at[pl.ds(srow, _SUP)], src_sc, sm).wait()
        pltpu.make_async_copy(dst_hbm.at[pl.ds(srow, _SUP)], dst_sc, sm).wait()
        pltpu.make_async_copy(w_hbm.at[pl.ds(srow, _SUP)], w_sc, sm).wait()

        gbufs = ((g0, sg0), (g1, sg1), (g2, sg2))
        sbufs = ((s0, ss0), (s1, ss1), (s2, ss2))
        for u in range(3):
            _g_start(u, *gbufs[u])
        for u in range(3):
            gb, gs = gbufs[u]
            sb, ssem = sbufs[u]
            _g_wait(u, gb, gs)
            _scale(u, gb, sb)
            _g_start(u + 3, gb, gs)
            _s_start(u, sb, ssem)

        def _tri(t, c2):
            for u in range(3):
                j = 3 * t + u
                gb, gs = gbufs[u]
                sb, ssem = sbufs[u]
                _g_wait(j, gb, gs)
                _s_wait(j - 3, sb, ssem)
                _scale(j, gb, sb)
                _g_start(j + 3, gb, gs)
                _s_start(j, sb, ssem)
            return c2

        lax.fori_loop(1, _NTRI - 1, _tri, 0)

        jl = _SUP - 3
        for u in range(3):
            gb, gs = gbufs[u]
            sb, ssem = sbufs[u]
            _g_wait(jl + u, gb, gs)
            _s_wait(jl + u - 3, sb, ssem)
            _scale(jl + u, gb, sb)
            _s_start(jl + u, sb, ssem)
        for u in range(3):
            sb, ssem = sbufs[u]
            _s_wait(jl + u, sb, ssem)
        return carry

    lax.fori_loop(0, _NSUP, _sup_body, 0)

    plsc.subcore_barrier()

    nflush = _N // _NS

    @pl.when(cid == 0)
    def _():
        pltpu.sync_copy(acc.at[pl.ds(sid * nflush, nflush)],
                        out0.at[pl.ds(sid * nflush, nflush)])

    @pl.when(cid == 1)
    def _():
        pltpu.sync_copy(acc.at[pl.ds(sid * nflush, nflush)],
                        out1.at[pl.ds(sid * nflush, nflush)])


_BLK = 400
_NBLK = _N // _BLK
_BLK_PER_W = 4


@functools.partial(
    pl.kernel,
    out_type=[
        jax.ShapeDtypeStruct((_N, _EMB), jnp.float32),
        jax.ShapeDtypeStruct((_N, _EMB), jnp.bfloat16),
    ],
    mesh=_mesh,
    compiler_params=pltpu.CompilerParams(use_tc_tiling_on_sc=False, needs_layout_passes=False),
    scratch_types=[
        pltpu.VMEM((_BLK, _EMB), jnp.float32),
        pltpu.VMEM((_BLK, _EMB), jnp.float32),
        pltpu.VMEM((_BLK, _EMB), jnp.bfloat16),
        pltpu.SemaphoreType.DMA,
    ],
)
def _add2(a_hbm, b_hbm, out, out_bf, abuf, bbuf, pbuf, sem):
    cid = lax.axis_index("c")
    sid = lax.axis_index("s")
    wid = sid * _NC + cid

    def _accum(r, c2):
        lo = abuf[r, pl.ds(0, 16)] + bbuf[r, pl.ds(0, 16)]
        hi = abuf[r, pl.ds(16, 16)] + bbuf[r, pl.ds(16, 16)]
        abuf[r, pl.ds(0, 16)] = lo
        abuf[r, pl.ds(16, 16)] = hi
        pbuf[r, pl.ds(0, _EMB)] = plsc.pack(
            lo, hi, format=plsc.PackFormat.INTERLEAVED)
        return c2

    for k in range(_BLK_PER_W):
        b = wid + _NW * k

        @pl.when(b < _NBLK)
        def _():
            off = b * _BLK
            pltpu.async_copy(a_hbm.at[pl.ds(off, _BLK)], abuf, sem)
            pltpu.async_copy(b_hbm.at[pl.ds(off, _BLK)], bbuf, sem)
            pltpu.make_async_copy(a_hbm.at[pl.ds(off, _BLK)], abuf, sem).wait()
            pltpu.make_async_copy(b_hbm.at[pl.ds(off, _BLK)], bbuf, sem).wait()
            lax.fori_loop(0, _BLK, _accum, 0, unroll=4)
            pltpu.async_copy(abuf, out.at[pl.ds(off, _BLK)], sem)
            pltpu.async_copy(pbuf, out_bf.at[pl.ds(off, _BLK)], sem)
            pltpu.make_async_copy(abuf, out.at[pl.ds(off, _BLK)], sem).wait()
            pltpu.make_async_copy(pbuf, out_bf.at[pl.ds(off, _BLK)],
                                  sem).wait()


@functools.partial(
    pl.kernel,
    out_type=jax.ShapeDtypeStruct((_N, _EMB), jnp.float32),
    mesh=_mesh,
    compiler_params=pltpu.CompilerParams(use_tc_tiling_on_sc=False, needs_layout_passes=False),
    scratch_types=[
        pltpu.VMEM((_BLK, _EMB), jnp.float32),
        pltpu.VMEM((_BLK, _EMB), jnp.float32),
        pltpu.VMEM((_BLK, _EMB), jnp.float32),
        pltpu.VMEM((_BLK, _EMB), jnp.float32),
        pltpu.SemaphoreType.DMA,
    ],
)
def _add4_mean(a_hbm, b_hbm, c_hbm, d_hbm, out, abuf, bbuf, cbuf, dbuf, sem):
    cid = lax.axis_index("c")
    sid = lax.axis_index("s")
    wid = sid * _NC + cid
    third = jnp.float32(1.0 / 3.0)

    def _accum(r, c2):
        lo = ((abuf[r, pl.ds(0, 16)] + bbuf[r, pl.ds(0, 16)])
              + (cbuf[r, pl.ds(0, 16)] + dbuf[r, pl.ds(0, 16)])) * third
        hi = ((abuf[r, pl.ds(16, 16)] + bbuf[r, pl.ds(16, 16)])
              + (cbuf[r, pl.ds(16, 16)] + dbuf[r, pl.ds(16, 16)])) * third
        abuf[r, pl.ds(0, 16)] = lo
        abuf[r, pl.ds(16, 16)] = hi
        return c2

    for k in range(_BLK_PER_W):
        b = wid + _NW * k

        @pl.when(b < _NBLK)
        def _():
            off = b * _BLK
            pltpu.async_copy(a_hbm.at[pl.ds(off, _BLK)], abuf, sem)
            pltpu.async_copy(b_hbm.at[pl.ds(off, _BLK)], bbuf, sem)
            pltpu.async_copy(c_hbm.at[pl.ds(off, _BLK)], cbuf, sem)
            pltpu.async_copy(d_hbm.at[pl.ds(off, _BLK)], dbuf, sem)
            pltpu.make_async_copy(a_hbm.at[pl.ds(off, _BLK)], abuf, sem).wait()
            pltpu.make_async_copy(b_hbm.at[pl.ds(off, _BLK)], bbuf, sem).wait()
            pltpu.make_async_copy(c_hbm.at[pl.ds(off, _BLK)], cbuf, sem).wait()
            pltpu.make_async_copy(d_hbm.at[pl.ds(off, _BLK)], dbuf, sem).wait()
            lax.fori_loop(0, _BLK, _accum, 0, unroll=4)
            pltpu.sync_copy(abuf, out.at[pl.ds(off, _BLK)])


def _to_packed_bf16(x):
    lo = x[:, : _EMB // 2]
    hi = x[:, _EMB // 2:]
    return jnp.stack([lo, hi], axis=-1).reshape(_N, _EMB).astype(jnp.bfloat16)


def kernel(edge_index, edge_weight, user_emb_w, item_emb_w):
    all0 = jnp.concatenate([user_emb_w, item_emb_w], axis=0)
    pad = _E_PAD - _E
    src = jnp.concatenate([edge_index[0], jnp.zeros((pad,), jnp.int32)])
    dst = jnp.concatenate([edge_index[1], jnp.zeros((pad,), jnp.int32)])
    w = jnp.concatenate([edge_weight, jnp.zeros((pad,), jnp.float32)])
    src2d = src.reshape(_E_PAD // _CH, _CH)
    dst2d = dst.reshape(_E_PAD // _CH, _CH)
    w2d = w.reshape(_E_PAD // _CH, _CH)

    p0, p1 = _layer(src2d, dst2d, w2d, _to_packed_bf16(all0))
    emb1, emb1_bf = _add2(p0, p1)
    q0, q1 = _layer(src2d, dst2d, w2d, emb1_bf)
    final = _add4_mean(all0, emb1, q0, q1)
    return final[:_N_USERS], final[_N_USERS:]
